# Initial kernel scaffold; baseline (speedup 1.0000x reference)
import jax
import jax.numpy as jnp
from jax.experimental import pallas as pl


def kernel(minutiae):
    raise NotImplementedError("write your pallas kernel here")


# speedup vs baseline: 70.3986x; 70.3986x over previous
import functools

import jax
import jax.numpy as jnp
import numpy as np
from jax.experimental import pallas as pl
from jax.experimental.pallas import tpu as pltpu

B = 128
DIST_T = np.float32(16.0)
ANG_T = np.float32(np.pi / 6.0)
TWO_PI = np.float32(2.0 * np.pi)


def _eye(n):
    r = jax.lax.broadcasted_iota(jnp.int32, (n, n), 0)
    c = jax.lax.broadcasted_iota(jnp.int32, (n, n), 1)
    return (r == c).astype(jnp.float32)


def _col2row(eye, v):
    return jnp.sum(eye * v, axis=0, keepdims=True)


def _rank_kernel(ntot, m_blk_ref, mT_ref, rank_col_ref, rank_row_ref):
    b = pl.program_id(0)
    s_c = m_blk_ref[:, 3:4]
    s_all = mT_ref[3:4, :]
    j_row = jax.lax.broadcasted_iota(jnp.int32, (1, ntot), 1)
    i_col = jax.lax.broadcasted_iota(jnp.int32, (B, 1), 0) + b * B
    gt = (s_all > s_c).astype(jnp.float32)
    tie = ((s_all == s_c) & (j_row < i_col)).astype(jnp.float32)
    rc = jnp.sum(gt + tie, axis=1, keepdims=True)
    rank_col_ref[:, :] = rc
    rank_row_ref[:, :] = _col2row(_eye(B), rc)


def _perm_kernel(ntot, rank_row_ref, rank_col_ref, m_ref, mT_ref,
                 sorted_ref, sortedT_ref):
    b = pl.program_id(0)
    base = (b * B).astype(jnp.float32)
    neg = jnp.float32(-1e30)

    rr = rank_row_ref[:, :]
    pos_c = base + jax.lax.broadcasted_iota(jnp.int32, (B, 1), 0).astype(
        jnp.float32
    )
    a_blk = rr == pos_c
    cols = []
    for c in range(4):
        vals = mT_ref[c : c + 1, :]
        cols.append(jnp.max(jnp.where(a_blk, vals, neg), axis=1, keepdims=True))
    sorted_ref[:, :] = jnp.concatenate(cols, axis=1)

    rc = rank_col_ref[:, :]
    pos_r = base + jax.lax.broadcasted_iota(jnp.int32, (1, B), 1).astype(
        jnp.float32
    )
    o_blk = rc == pos_r
    rows = []
    for c in range(4):
        valc = m_ref[:, c : c + 1]
        rows.append(jnp.max(jnp.where(o_blk, valc, neg), axis=0, keepdims=True))
    sortedT_ref[:, :] = jnp.concatenate(rows, axis=0)


def _conf(dx, dy, da):
    dist = jnp.sqrt(dx * dx + dy * dy)
    ad = jnp.abs(da)
    am = jnp.minimum(ad, TWO_PI - ad)
    return ((dist < DIST_T) & (am < ANG_T)).astype(jnp.float32)


def _nms_kernel(ntot, sorted_full_ref, sorted_blk_ref, sortedT_blk_ref,
                out_ref, keep_ref):
    b = pl.program_id(0)

    @pl.when(b == 0)
    def _():
        keep_ref[:, :] = jnp.ones((ntot, 1), jnp.float32)

    eye = _eye(B)
    x_all = sorted_full_ref[:, 0:1]
    y_all = sorted_full_ref[:, 1:2]
    a_all = sorted_full_ref[:, 2:3]
    xb_r = sortedT_blk_ref[0:1, :]
    yb_r = sortedT_blk_ref[1:2, :]
    ab_r = sortedT_blk_ref[2:3, :]
    xb_c = sorted_blk_ref[:, 0:1]
    yb_c = sorted_blk_ref[:, 1:2]
    ab_c = sorted_blk_ref[:, 2:3]

    conf2 = _conf(x_all - xb_r, y_all - yb_r, a_all - ab_r)
    confb = _conf(xb_c - xb_r, yb_c - yb_r, ab_c - ab_r)
    tri = (
        jax.lax.broadcasted_iota(jnp.int32, (B, B), 0)
        > jax.lax.broadcasted_iota(jnp.int32, (B, B), 1)
    ).astype(jnp.float32)
    mb = confb * tri

    kb = keep_ref[pl.ds(b * B, B), :]
    und_c0 = kb
    kp_c0 = jnp.zeros((B, 1), jnp.float32)
    und_r0 = _col2row(eye, und_c0)
    kp_r0 = jnp.zeros((1, B), jnp.float32)

    def cond_f(st):
        return jnp.sum(st[0]) > 0.0

    def body_f(st):
        und_c, und_r, kp_c, kp_r = st
        q_r = kp_r + und_r
        blocked = jnp.max(mb * q_r, axis=1, keepdims=True)
        suppc = jnp.max(mb * kp_r, axis=1, keepdims=True)
        nk = und_c * (1.0 - blocked)
        ns = und_c * suppc
        und_c2 = und_c - nk - ns
        kp_c2 = kp_c + nk
        return (und_c2, _col2row(eye, und_c2), kp_c2, kp_r + _col2row(eye, nk))

    _, _, kp_c, kp_r = jax.lax.while_loop(
        cond_f, body_f, (und_c0, und_r0, kp_c0, kp_r0)
    )

    out_ref[:, :] = sorted_blk_ref[:, :] * kp_c

    j_col = jax.lax.broadcasted_iota(jnp.int32, (ntot, 1), 0)
    later = (j_col >= (b + 1) * B).astype(jnp.float32)
    supp = jnp.max(conf2 * kp_r, axis=1, keepdims=True)
    keep_ref[:, :] = keep_ref[:, :] * (1.0 - supp * later)


def kernel(minutiae):
    n0 = minutiae.shape[0]
    ntot = ((n0 + B - 1) // B) * B
    nb = ntot // B
    m = minutiae.astype(jnp.float32)
    if ntot > n0:
        pad = jnp.broadcast_to(
            jnp.array([1e9, 1e9, 0.0, -1.0], jnp.float32), (ntot - n0, 4)
        )
        m = jnp.concatenate([m, pad], axis=0)
    mT = m.T

    f32 = jnp.float32
    rank_col, rank_row = pl.pallas_call(
        functools.partial(_rank_kernel, ntot),
        grid=(nb,),
        in_specs=[
            pl.BlockSpec((B, 4), lambda b: (b, 0)),
            pl.BlockSpec((4, ntot), lambda b: (0, 0)),
        ],
        out_specs=[
            pl.BlockSpec((B, 1), lambda b: (b, 0)),
            pl.BlockSpec((1, B), lambda b: (0, b)),
        ],
        out_shape=[
            jax.ShapeDtypeStruct((ntot, 1), f32),
            jax.ShapeDtypeStruct((1, ntot), f32),
        ],
    )(m, mT)

    sorted_m, sorted_t = pl.pallas_call(
        functools.partial(_perm_kernel, ntot),
        grid=(nb,),
        in_specs=[
            pl.BlockSpec((1, ntot), lambda b: (0, 0)),
            pl.BlockSpec((ntot, 1), lambda b: (0, 0)),
            pl.BlockSpec((ntot, 4), lambda b: (0, 0)),
            pl.BlockSpec((4, ntot), lambda b: (0, 0)),
        ],
        out_specs=[
            pl.BlockSpec((B, 4), lambda b: (b, 0)),
            pl.BlockSpec((4, B), lambda b: (0, b)),
        ],
        out_shape=[
            jax.ShapeDtypeStruct((ntot, 4), f32),
            jax.ShapeDtypeStruct((4, ntot), f32),
        ],
    )(rank_row, rank_col, m, mT)

    out = pl.pallas_call(
        functools.partial(_nms_kernel, ntot),
        grid=(nb,),
        in_specs=[
            pl.BlockSpec((ntot, 4), lambda b: (0, 0)),
            pl.BlockSpec((B, 4), lambda b: (b, 0)),
            pl.BlockSpec((4, B), lambda b: (0, b)),
        ],
        out_specs=pl.BlockSpec((B, 4), lambda b: (b, 0)),
        out_shape=jax.ShapeDtypeStruct((ntot, 4), f32),
        scratch_shapes=[pltpu.VMEM((ntot, 1), f32)],
    )(sorted_m, sorted_m, sorted_t)

    return out[:n0]


# speedup vs baseline: 130.4052x; 1.8524x over previous
import functools

import jax
import jax.numpy as jnp
import numpy as np
from jax.experimental import pallas as pl
from jax.experimental.pallas import tpu as pltpu

B = 128
DIST2_T = np.float32(256.0)
ANG_T = np.float32(np.pi / 6.0)
TWO_PI = np.float32(2.0 * np.pi)


def _eye(n):
    r = jax.lax.broadcasted_iota(jnp.int32, (n, n), 0)
    c = jax.lax.broadcasted_iota(jnp.int32, (n, n), 1)
    return (r == c).astype(jnp.float32)


def _col2row(eye, v):
    return jnp.sum(eye * v, axis=0, keepdims=True)


def _row2col(eye, v):
    return jnp.sum(eye * v, axis=1, keepdims=True)


def _rank_kernel(ntot, m_ref, mT_blk_ref, rank_row_ref):
    b = pl.program_id(0)
    s_all_c = m_ref[:, 3:4]
    s_i_r = mT_blk_ref[3:4, :]
    j_row = jax.lax.broadcasted_iota(jnp.int32, (ntot, 1), 0)
    i_col = jax.lax.broadcasted_iota(jnp.int32, (1, B), 1) + b * B
    gt = (s_all_c > s_i_r).astype(jnp.float32)
    tie = ((s_all_c == s_i_r) & (j_row < i_col)).astype(jnp.float32)
    rank_row_ref[:, :] = jnp.sum(gt + tie, axis=0, keepdims=True)


def _perm_kernel(ntot, rank_row_ref, mT_ref, sorted_ref, sortedT_ref):
    b = pl.program_id(0)
    base = (b * B).astype(jnp.float32)
    neg = jnp.float32(-1e30)
    eye = _eye(B)

    rr = rank_row_ref[:, :]
    pos_c = base + jax.lax.broadcasted_iota(jnp.int32, (B, 1), 0).astype(
        jnp.float32
    )
    a_blk = rr == pos_c
    cols = []
    for c in range(4):
        vals = mT_ref[c : c + 1, :]
        col = jnp.max(jnp.where(a_blk, vals, neg), axis=1, keepdims=True)
        cols.append(col)
        sortedT_ref[c : c + 1, :] = _col2row(eye, col)
    sorted_ref[:, :] = jnp.concatenate(cols, axis=1)


def _nms_kernel(ntot, sortedT_full_ref, sorted_blk_ref, sortedT_blk_ref,
                out_ref, keep_ref):
    b = pl.program_id(0)

    @pl.when(b == 0)
    def _():
        keep_ref[:, :] = jnp.ones((1, ntot), jnp.float32)

    eye = _eye(B)
    x_all = sortedT_full_ref[0:1, :]
    y_all = sortedT_full_ref[1:2, :]
    a_all = sortedT_full_ref[2:3, :]
    xb_r = sortedT_blk_ref[0:1, :]
    yb_r = sortedT_blk_ref[1:2, :]
    ab_r = sortedT_blk_ref[2:3, :]
    xb_c = sorted_blk_ref[:, 0:1]
    yb_c = sorted_blk_ref[:, 1:2]
    ab_c = sorted_blk_ref[:, 2:3]

    def conf(dx, dy, da):
        d2 = dx * dx + dy * dy
        ad = jnp.abs(da)
        am = jnp.minimum(ad, TWO_PI - ad)
        return ((d2 < DIST2_T) & (am < ANG_T)).astype(jnp.float32)

    conf2 = conf(xb_c - x_all, yb_c - y_all, ab_c - a_all)
    confb = conf(xb_c - xb_r, yb_c - yb_r, ab_c - ab_r)
    tri = (
        jax.lax.broadcasted_iota(jnp.int32, (B, B), 0)
        < jax.lax.broadcasted_iota(jnp.int32, (B, B), 1)
    ).astype(jnp.float32)
    mb = confb * tri

    kb_r = keep_ref[0:1, pl.ds(b * B, B)]
    und_r0 = kb_r
    und_c0 = _row2col(eye, kb_r)
    kp_r0 = jnp.zeros((1, B), jnp.float32)
    kp_c0 = jnp.zeros((B, 1), jnp.float32)

    def cond_f(st):
        return jnp.sum(st[0]) > 0.0

    def body_f(st):
        und_r, und_c, kp_r, kp_c = st
        q_c = kp_c + und_c
        blocked = jnp.max(mb * q_c, axis=0, keepdims=True)
        suppb = jnp.max(mb * kp_c, axis=0, keepdims=True)
        nk = und_r * (1.0 - blocked)
        ns = und_r * suppb
        und_r2 = und_r - nk - ns
        kp_r2 = kp_r + nk
        return (und_r2, _row2col(eye, und_r2), kp_r2, kp_c + _row2col(eye, nk))

    _, _, kp_r, kp_c = jax.lax.while_loop(
        cond_f, body_f, (und_r0, und_c0, kp_r0, kp_c0)
    )

    out_ref[:, :] = sorted_blk_ref[:, :] * kp_c

    j_lane = jax.lax.broadcasted_iota(jnp.int32, (1, ntot), 1)
    later = (j_lane >= (b + 1) * B).astype(jnp.float32)
    supp = jnp.max(conf2 * kp_c, axis=0, keepdims=True)
    keep_ref[:, :] = keep_ref[:, :] * (1.0 - supp * later)


def kernel(minutiae):
    n0 = minutiae.shape[0]
    ntot = ((n0 + B - 1) // B) * B
    nb = ntot // B
    m = minutiae.astype(jnp.float32)
    if ntot > n0:
        pad = jnp.broadcast_to(
            jnp.array([1e9, 1e9, 0.0, -1.0], jnp.float32), (ntot - n0, 4)
        )
        m = jnp.concatenate([m, pad], axis=0)
    mT = m.T

    f32 = jnp.float32
    rank_row = pl.pallas_call(
        functools.partial(_rank_kernel, ntot),
        grid=(nb,),
        in_specs=[
            pl.BlockSpec((ntot, 4), lambda b: (0, 0)),
            pl.BlockSpec((4, B), lambda b: (0, b)),
        ],
        out_specs=pl.BlockSpec((1, B), lambda b: (0, b)),
        out_shape=jax.ShapeDtypeStruct((1, ntot), f32),
    )(m, mT)

    sorted_m, sorted_t = pl.pallas_call(
        functools.partial(_perm_kernel, ntot),
        grid=(nb,),
        in_specs=[
            pl.BlockSpec((1, ntot), lambda b: (0, 0)),
            pl.BlockSpec((4, ntot), lambda b: (0, 0)),
        ],
        out_specs=[
            pl.BlockSpec((B, 4), lambda b: (b, 0)),
            pl.BlockSpec((4, B), lambda b: (0, b)),
        ],
        out_shape=[
            jax.ShapeDtypeStruct((ntot, 4), f32),
            jax.ShapeDtypeStruct((4, ntot), f32),
        ],
    )(rank_row, mT)

    out = pl.pallas_call(
        functools.partial(_nms_kernel, ntot),
        grid=(nb,),
        in_specs=[
            pl.BlockSpec((4, ntot), lambda b: (0, 0)),
            pl.BlockSpec((B, 4), lambda b: (b, 0)),
            pl.BlockSpec((4, B), lambda b: (0, b)),
        ],
        out_specs=pl.BlockSpec((B, 4), lambda b: (b, 0)),
        out_shape=jax.ShapeDtypeStruct((ntot, 4), f32),
        scratch_shapes=[pltpu.VMEM((1, ntot), f32)],
    )(sorted_t, sorted_m, sorted_t)

    return out[:n0]


# speedup vs baseline: 175.2085x; 1.3436x over previous
import functools

import jax
import jax.numpy as jnp
import numpy as np
from jax.experimental import pallas as pl
from jax.experimental.pallas import tpu as pltpu
from jax.experimental.pallas import tpu_sc as plsc

B = 128
DIST2_T = np.float32(256.0)
ANG_T = np.float32(np.pi / 6.0)
TWO_PI = np.float32(2.0 * np.pi)
NWORKERS = 32
SC_LANES = 16


def _eye(n):
    r = jax.lax.broadcasted_iota(jnp.int32, (n, n), 0)
    c = jax.lax.broadcasted_iota(jnp.int32, (n, n), 1)
    return (r == c).astype(jnp.float32)


def _row2col(eye, v):
    return jnp.sum(eye * v, axis=1, keepdims=True)


def _rank_kernel(ntot, m_ref, mT_blk_ref, rank_row_ref):
    b = pl.program_id(0)
    s_all_c = m_ref[:, 3:4]
    s_i_r = mT_blk_ref[3:4, :]
    j_row = jax.lax.broadcasted_iota(jnp.int32, (ntot, 1), 0)
    i_col = jax.lax.broadcasted_iota(jnp.int32, (1, B), 1) + b * B
    gt = (s_all_c > s_i_r).astype(jnp.float32)
    tie = ((s_all_c == s_i_r) & (j_row < i_col)).astype(jnp.float32)
    rank_row_ref[:, :] = jnp.sum(gt + tie, axis=0, keepdims=True)


def _src_kernel(ntot, rank_row_ref, src_ref):
    b = pl.program_id(0)
    base = (b * B).astype(jnp.float32)
    rr = rank_row_ref[:, :]
    pos_c = base + jax.lax.broadcasted_iota(jnp.int32, (B, 1), 0).astype(
        jnp.float32
    )
    a_blk = rr == pos_c
    j_row = jax.lax.broadcasted_iota(jnp.int32, (1, ntot), 1).astype(jnp.float32)
    src_ref[:, :] = jnp.max(
        jnp.where(a_blk, j_row, jnp.float32(-1.0)), axis=1, keepdims=True
    )


def _make_sc_gather(ntot):
    chunk = ntot // NWORKERS
    mesh = plsc.VectorSubcoreMesh(core_axis_name="c", subcore_axis_name="s")

    @functools.partial(
        pl.kernel,
        mesh=mesh,
        out_type=jax.ShapeDtypeStruct((4 * ntot,), jnp.float32),
        compiler_params=pltpu.CompilerParams(needs_layout_passes=False),
        scratch_types=[
            pltpu.VMEM((4 * ntot,), jnp.float32),
            pltpu.VMEM((chunk,), jnp.int32),
            pltpu.VMEM((4 * chunk,), jnp.float32),
        ],
    )
    def sc_gather(mT_hbm, src_hbm, outT_hbm, cols_v, idx_v, vals_v):
        wid = jax.lax.axis_index("s") * 2 + jax.lax.axis_index("c")
        base = wid * chunk
        pltpu.sync_copy(mT_hbm, cols_v)
        pltpu.sync_copy(src_hbm.at[pl.ds(base, chunk)], idx_v)
        for g in range(chunk // SC_LANES):
            idx = idx_v[pl.ds(g * SC_LANES, SC_LANES)]
            for c in range(4):
                vals = plsc.load_gather(cols_v, [idx + c * ntot])
                vals_v[pl.ds(c * chunk + g * SC_LANES, SC_LANES)] = vals
        for c in range(4):
            pltpu.sync_copy(
                vals_v.at[pl.ds(c * chunk, chunk)],
                outT_hbm.at[pl.ds(c * ntot + base, chunk)],
            )

    return sc_gather


def _nms_kernel(ntot, sortedT_full_ref, sortedT_blk_ref, out_ref, keep_ref):
    b = pl.program_id(0)

    @pl.when(b == 0)
    def _():
        keep_ref[:, :] = jnp.ones((1, ntot), jnp.float32)

    eye = _eye(B)
    x_all = sortedT_full_ref[0:1, :]
    y_all = sortedT_full_ref[1:2, :]
    a_all = sortedT_full_ref[2:3, :]
    xb_r = sortedT_blk_ref[0:1, :]
    yb_r = sortedT_blk_ref[1:2, :]
    ab_r = sortedT_blk_ref[2:3, :]
    cols = [
        _row2col(eye, sortedT_blk_ref[c : c + 1, :]) for c in range(4)
    ]
    xb_c, yb_c, ab_c = cols[0], cols[1], cols[2]

    def conf(dx, dy, da):
        d2 = dx * dx + dy * dy
        ad = jnp.abs(da)
        am = jnp.minimum(ad, TWO_PI - ad)
        return ((d2 < DIST2_T) & (am < ANG_T)).astype(jnp.float32)

    confb = conf(xb_c - xb_r, yb_c - yb_r, ab_c - ab_r)
    tri = (
        jax.lax.broadcasted_iota(jnp.int32, (B, B), 0)
        < jax.lax.broadcasted_iota(jnp.int32, (B, B), 1)
    ).astype(jnp.float32)
    mb = confb * tri

    kb_r = keep_ref[0:1, pl.ds(b * B, B)]
    und_r0 = kb_r
    und_c0 = _row2col(eye, kb_r)
    kp_r0 = jnp.zeros((1, B), jnp.float32)
    kp_c0 = jnp.zeros((B, 1), jnp.float32)

    def cond_f(st):
        return jnp.sum(st[0]) > 0.0

    def body_f(st):
        und_r, und_c, kp_r, kp_c = st
        q_c = kp_c + und_c
        blocked = jnp.max(mb * q_c, axis=0, keepdims=True)
        suppb = jnp.max(mb * kp_c, axis=0, keepdims=True)
        nk = und_r * (1.0 - blocked)
        ns = und_r * suppb
        und_r2 = und_r - nk - ns
        kp_r2 = kp_r + nk
        return (und_r2, _row2col(eye, und_r2), kp_r2, kp_c + _row2col(eye, nk))

    _, _, kp_r, kp_c = jax.lax.while_loop(
        cond_f, body_f, (und_r0, und_c0, kp_r0, kp_c0)
    )

    out_ref[:, :] = jnp.concatenate(cols, axis=1) * kp_c

    conf2 = conf(xb_c - x_all, yb_c - y_all, ab_c - a_all)
    supp = jax.lax.dot_general(
        kp_r, conf2, (((1,), (0,)), ((), ())),
        preferred_element_type=jnp.float32,
    )
    j_lane = jax.lax.broadcasted_iota(jnp.int32, (1, ntot), 1)
    later = (j_lane >= (b + 1) * B).astype(jnp.float32)
    keep_ref[:, :] = keep_ref[:, :] * (1.0 - (supp > 0.0).astype(jnp.float32) * later)


def kernel(minutiae):
    n0 = minutiae.shape[0]
    ntot = ((n0 + B - 1) // B) * B
    nb = ntot // B
    m = minutiae.astype(jnp.float32)
    if ntot > n0:
        pad = jnp.broadcast_to(
            jnp.array([1e9, 1e9, 0.0, -1.0], jnp.float32), (ntot - n0, 4)
        )
        m = jnp.concatenate([m, pad], axis=0)
    mT = m.T

    f32 = jnp.float32
    rank_row = pl.pallas_call(
        functools.partial(_rank_kernel, ntot),
        grid=(nb,),
        in_specs=[
            pl.BlockSpec((ntot, 4), lambda b: (0, 0)),
            pl.BlockSpec((4, B), lambda b: (0, b)),
        ],
        out_specs=pl.BlockSpec((1, B), lambda b: (0, b)),
        out_shape=jax.ShapeDtypeStruct((1, ntot), f32),
    )(m, mT)

    src_col = pl.pallas_call(
        functools.partial(_src_kernel, ntot),
        grid=(nb,),
        in_specs=[pl.BlockSpec((1, ntot), lambda b: (0, 0))],
        out_specs=pl.BlockSpec((B, 1), lambda b: (b, 0)),
        out_shape=jax.ShapeDtypeStruct((ntot, 1), f32),
    )(rank_row)

    src_i32 = src_col.reshape(ntot).astype(jnp.int32)
    mT_flat = mT.reshape(4 * ntot)
    sorted_t = _make_sc_gather(ntot)(mT_flat, src_i32).reshape(4, ntot)

    out = pl.pallas_call(
        functools.partial(_nms_kernel, ntot),
        grid=(nb,),
        in_specs=[
            pl.BlockSpec((4, ntot), lambda b: (0, 0)),
            pl.BlockSpec((4, B), lambda b: (0, b)),
        ],
        out_specs=pl.BlockSpec((B, 4), lambda b: (b, 0)),
        out_shape=jax.ShapeDtypeStruct((ntot, 4), f32),
        scratch_shapes=[pltpu.VMEM((1, ntot), f32)],
    )(sorted_t, sorted_t)

    return out[:n0]


# speedup vs baseline: 192.3301x; 1.0977x over previous
import functools

import jax
import jax.numpy as jnp
import numpy as np
from jax.experimental import pallas as pl
from jax.experimental.pallas import tpu as pltpu
from jax.experimental.pallas import tpu_sc as plsc

B = 128
DIST2_T = np.float32(256.0)
ANG_T = np.float32(np.pi / 6.0)
TWO_PI = np.float32(2.0 * np.pi)
NWORKERS = 32
SC_LANES = 16


def _eye(n):
    r = jax.lax.broadcasted_iota(jnp.int32, (n, n), 0)
    c = jax.lax.broadcasted_iota(jnp.int32, (n, n), 1)
    return (r == c).astype(jnp.float32)


def _col2row(eye, v):
    return jnp.sum(eye * v, axis=0, keepdims=True)


def _row2col(eye, v):
    return jnp.sum(eye * v, axis=1, keepdims=True)


def _rank_kernel(ntot, m_ref, m_blk_ref, rank_row_ref):
    b = pl.program_id(0)
    s_all_c = m_ref[:, 3:4]
    s_i_r = _col2row(_eye(B), m_blk_ref[:, 3:4])
    j_row = jax.lax.broadcasted_iota(jnp.int32, (ntot, 1), 0)
    i_col = jax.lax.broadcasted_iota(jnp.int32, (1, B), 1) + b * B
    gt = (s_all_c > s_i_r).astype(jnp.float32)
    tie = ((s_all_c == s_i_r) & (j_row < i_col)).astype(jnp.float32)
    rank_row_ref[:, :] = jnp.sum(gt + tie, axis=0, keepdims=True).astype(
        jnp.int32
    )


def _make_sc_permute(ntot):
    chunk = ntot // NWORKERS
    ngroups = ntot // SC_LANES
    mesh = plsc.VectorSubcoreMesh(core_axis_name="c", subcore_axis_name="s")

    @functools.partial(
        pl.kernel,
        mesh=mesh,
        out_type=jax.ShapeDtypeStruct((4 * ntot,), jnp.float32),
        compiler_params=pltpu.CompilerParams(needs_layout_passes=False),
        scratch_types=[
            pltpu.VMEM((4 * ntot,), jnp.float32),
            pltpu.VMEM((ntot,), jnp.int32),
            pltpu.VMEM((chunk,), jnp.int32),
            pltpu.VMEM((4 * chunk,), jnp.float32),
        ],
    )
    def sc_permute(m_hbm, rank_hbm, outT_hbm, pts_v, rank_v, src_v, vals_v):
        wid = jax.lax.axis_index("s") * 2 + jax.lax.axis_index("c")
        base = wid * chunk
        pltpu.sync_copy(m_hbm, pts_v)
        pltpu.sync_copy(rank_hbm, rank_v)
        lane = jax.lax.broadcasted_iota(jnp.int32, (SC_LANES,), 0)
        for g in range(ngroups):
            r16 = rank_v[pl.ds(g * SC_LANES, SC_LANES)]
            loc = r16 - base
            mask = (loc >= 0) & (loc < chunk)
            plsc.store_scatter(src_v, [loc], lane + g * SC_LANES, mask=mask)
        for g in range(chunk // SC_LANES):
            s16 = src_v[pl.ds(g * SC_LANES, SC_LANES)]
            flat = s16 * 4
            for c in range(4):
                vals = plsc.load_gather(pts_v, [flat + c])
                vals_v[pl.ds(c * chunk + g * SC_LANES, SC_LANES)] = vals
        for c in range(4):
            pltpu.sync_copy(
                vals_v.at[pl.ds(c * chunk, chunk)],
                outT_hbm.at[pl.ds(c * ntot + base, chunk)],
            )

    return sc_permute


def _nms_kernel(ntot, sortedT_full_ref, sortedT_blk_ref, out_ref, keep_ref):
    b = pl.program_id(0)

    @pl.when(b == 0)
    def _():
        keep_ref[:, :] = jnp.ones((1, ntot), jnp.float32)

    eye = _eye(B)
    x_all = sortedT_full_ref[0:1, :]
    y_all = sortedT_full_ref[1:2, :]
    a_all = sortedT_full_ref[2:3, :]
    xb_r = sortedT_blk_ref[0:1, :]
    yb_r = sortedT_blk_ref[1:2, :]
    ab_r = sortedT_blk_ref[2:3, :]
    cols = [
        _row2col(eye, sortedT_blk_ref[c : c + 1, :]) for c in range(4)
    ]
    xb_c, yb_c, ab_c = cols[0], cols[1], cols[2]

    def conf(dx, dy, da):
        d2 = dx * dx + dy * dy
        ad = jnp.abs(da)
        am = jnp.minimum(ad, TWO_PI - ad)
        return ((d2 < DIST2_T) & (am < ANG_T)).astype(jnp.float32)

    confb = conf(xb_c - xb_r, yb_c - yb_r, ab_c - ab_r)
    tri = (
        jax.lax.broadcasted_iota(jnp.int32, (B, B), 0)
        < jax.lax.broadcasted_iota(jnp.int32, (B, B), 1)
    ).astype(jnp.float32)
    mb = confb * tri

    kb_r = keep_ref[0:1, pl.ds(b * B, B)]
    und_r0 = kb_r
    und_c0 = _row2col(eye, kb_r)
    kp_r0 = jnp.zeros((1, B), jnp.float32)
    kp_c0 = jnp.zeros((B, 1), jnp.float32)

    def cond_f(st):
        return jnp.sum(st[0]) > 0.0

    def body_f(st):
        und_r, und_c, kp_r, kp_c = st
        q_c = kp_c + und_c
        blocked = jnp.max(mb * q_c, axis=0, keepdims=True)
        suppb = jnp.max(mb * kp_c, axis=0, keepdims=True)
        nk = und_r * (1.0 - blocked)
        ns = und_r * suppb
        und_r2 = und_r - nk - ns
        kp_r2 = kp_r + nk
        return (und_r2, _row2col(eye, und_r2), kp_r2, kp_c + _row2col(eye, nk))

    _, _, kp_r, kp_c = jax.lax.while_loop(
        cond_f, body_f, (und_r0, und_c0, kp_r0, kp_c0)
    )

    out_ref[:, :] = jnp.concatenate(cols, axis=1) * kp_c

    conf2 = conf(xb_c - x_all, yb_c - y_all, ab_c - a_all)
    supp = jax.lax.dot_general(
        kp_r, conf2, (((1,), (0,)), ((), ())),
        preferred_element_type=jnp.float32,
    )
    keep_ref[:, :] = keep_ref[:, :] * (1.0 - (supp > 0.0).astype(jnp.float32))


def kernel(minutiae):
    n0 = minutiae.shape[0]
    ntot = ((n0 + B - 1) // B) * B
    nb = ntot // B
    m = minutiae.astype(jnp.float32)
    if ntot > n0:
        pad = jnp.broadcast_to(
            jnp.array([1e9, 1e9, 0.0, -1.0], jnp.float32), (ntot - n0, 4)
        )
        m = jnp.concatenate([m, pad], axis=0)

    f32 = jnp.float32
    rank_row = pl.pallas_call(
        functools.partial(_rank_kernel, ntot),
        grid=(nb,),
        in_specs=[
            pl.BlockSpec((ntot, 4), lambda b: (0, 0)),
            pl.BlockSpec((B, 4), lambda b: (b, 0)),
        ],
        out_specs=pl.BlockSpec((1, B), lambda b: (0, b)),
        out_shape=jax.ShapeDtypeStruct((1, ntot), jnp.int32),
    )(m, m)

    m_flat = m.reshape(4 * ntot)
    rank_flat = rank_row.reshape(ntot)
    sorted_t = _make_sc_permute(ntot)(m_flat, rank_flat).reshape(4, ntot)

    out = pl.pallas_call(
        functools.partial(_nms_kernel, ntot),
        grid=(nb,),
        in_specs=[
            pl.BlockSpec((4, ntot), lambda b: (0, 0)),
            pl.BlockSpec((4, B), lambda b: (0, b)),
        ],
        out_specs=pl.BlockSpec((B, 4), lambda b: (b, 0)),
        out_shape=jax.ShapeDtypeStruct((ntot, 4), f32),
        scratch_shapes=[pltpu.VMEM((1, ntot), f32)],
    )(sorted_t, sorted_t)

    return out[:n0]
